# baseline (speedup 1.0000x reference)
import functools

import jax
import jax.numpy as jnp
from jax import lax
from jax.experimental import pallas as pl
from jax.experimental.pallas import tpu as pltpu
from jax.experimental.pallas import tpu_sc as plsc

ROW = 128
GROUP_ROWS = 2
GS = ROW * GROUP_ROWS
SB_ROWS = 8
SB = ROW * SB_ROWS
N_GROUPS = SB // GS
NBUF = 2


@functools.partial(jax.jit, static_argnums=(3, 4, 5))
def _embed(xp2, xh2, table2, N, NC, NS):
    D2 = table2.shape[1]
    D = D2 // 2
    NV = D // 16
    NW = NC * NS
    per_w = N // NW
    n_sb = per_w // SB
    mesh = plsc.VectorSubcoreMesh(core_axis_name="c", subcore_axis_name="s")

    @functools.partial(
        pl.kernel,
        mesh=mesh,
        out_type=jax.ShapeDtypeStruct((N // 2, D2), jnp.float32),
        compiler_params=pltpu.CompilerParams(use_tc_tiling_on_sc=True),
        scratch_types=[
            pltpu.VMEM((SB_ROWS, ROW), jnp.int32),
            pltpu.VMEM((SB_ROWS, ROW), jnp.int32),
            pltpu.VMEM((NBUF, GS, D2), jnp.float32),
            pltpu.VMEM((NBUF, GS // 2, D2), jnp.float32),
            pltpu.SemaphoreType.DMA,
            pltpu.SemaphoreType.DMA,
        ],
    )
    def k(xp_hbm, xh_hbm, table_hbm, out_hbm, pid_v, h_v, pair_v, out_v,
          sem0, sem1):
        wid = lax.axis_index("s") * NC + lax.axis_index("c")
        sems = [sem0, sem1]

        def issue(g, b):
            for r in range(GROUP_ROWS):
                pltpu.async_copy(
                    table_hbm.at[pid_v.at[g * GROUP_ROWS + r]],
                    pair_v.at[b, pl.ds(r * ROW, ROW)],
                    sems[b],
                )

        def drain(sb, g, b):
            pltpu.make_async_copy(
                table_hbm.at[pl.ds(0, GS)], pair_v.at[b], sems[b]
            ).wait()

            def extract(i, carry):
                jj0 = g * GS + i * 16
                hvec = h_v[jj0 >> 7, pl.ds(jj0 & (ROW - 1), 16)]
                for l in range(16):
                    j = i * 16 + l
                    src0 = hvec[l] * D
                    dst0 = (l & 1) * D
                    q = j >> 1
                    for t in range(NV):
                        out_v[b, q, pl.ds(dst0 + t * 16, 16)] = (
                            pair_v[b, j, pl.ds(src0 + t * 16, 16)])
                return carry

            lax.fori_loop(0, GS // 16, extract, 0)
            obase = pl.multiple_of(
                (wid * per_w + sb * SB + g * GS) // 2, GS // 2)
            pltpu.sync_copy(out_v.at[b], out_hbm.at[pl.ds(obase, GS // 2)])

        def body(sb, carry):
            sbrow = pl.multiple_of((wid * per_w + sb * SB) // ROW, SB_ROWS)
            pltpu.sync_copy(xp_hbm.at[pl.ds(sbrow, SB_ROWS)], pid_v)
            pltpu.sync_copy(xh_hbm.at[pl.ds(sbrow, SB_ROWS)], h_v)
            issue(0, 0)
            issue(1, 1)
            drain(sb, 0, 0)
            issue(2, 0)
            drain(sb, 1, 1)
            issue(3, 1)
            drain(sb, 2, 0)
            drain(sb, 3, 1)
            return carry

        lax.fori_loop(0, n_sb, body, 0)

    return k(xp2, xh2, table2)


def kernel(x, table):
    B, H = x.shape
    V, D = table.shape
    N = B * H
    info = plsc.get_sparse_core_info()
    NC, NS = info.num_cores, info.num_subcores
    assert N % (NC * NS * SB) == 0 and V % 2 == 0 and D == 64
    xi = x.astype(jnp.int32)
    xp2 = (xi >> 1).reshape(N // ROW, ROW)
    xh2 = (xi & 1).reshape(N // ROW, ROW)
    table2 = table.reshape(V // 2, 2 * D)
    out = _embed(xp2, xh2, table2, N, NC, NS)
    return out.reshape(B, H, D)


# speedup vs baseline: 1.3198x; 1.3198x over previous
import functools

import jax
import jax.numpy as jnp
from jax import lax
from jax.experimental import pallas as pl
from jax.experimental.pallas import tpu as pltpu
from jax.experimental.pallas import tpu_sc as plsc

ROW = 128
GROUP_ROWS = 4
GS = ROW * GROUP_ROWS
NBUF = 2


@functools.partial(jax.jit, static_argnums=(2, 3, 4))
def _embed(x2, table, N, NC, NS):
    D = table.shape[1]
    NW = NC * NS
    per_w = N // NW
    n_groups = per_w // GS
    assert n_groups % 2 == 0 and n_groups >= 4
    mesh = plsc.VectorSubcoreMesh(core_axis_name="c", subcore_axis_name="s")

    @functools.partial(
        pl.kernel,
        mesh=mesh,
        out_type=jax.ShapeDtypeStruct((N, D), jnp.float32),
        compiler_params=pltpu.CompilerParams(use_tc_tiling_on_sc=False),
        scratch_types=[
            pltpu.VMEM((NBUF, GROUP_ROWS, ROW), jnp.int32),
            pltpu.VMEM((NBUF, GS, D), jnp.float32),
            pltpu.SemaphoreType.DMA,
            pltpu.SemaphoreType.DMA,
        ],
    )
    def k(x_hbm, table_hbm, out_hbm, idx_v, rows_v, sem0, sem1):
        wid = lax.axis_index("s") * NC + lax.axis_index("c")
        wb = wid * per_w
        sems = [sem0, sem1]

        def issue(g, b):
            base = pl.multiple_of(wb + g * GS, GS)
            row0 = pl.multiple_of(base // ROW, GROUP_ROWS)
            pltpu.sync_copy(x_hbm.at[pl.ds(row0, GROUP_ROWS)], idx_v.at[b])
            for j in range(GROUP_ROWS):
                pltpu.async_copy(
                    table_hbm.at[idx_v.at[b, j]],
                    rows_v.at[b, pl.ds(j * ROW, ROW)],
                    sems[b],
                )

        def drain(g, b):
            pltpu.make_async_copy(
                out_hbm.at[pl.ds(0, GS)], rows_v.at[b], sems[b]
            ).wait()
            base = pl.multiple_of(wb + g * GS, GS)
            pltpu.sync_copy(rows_v.at[b], out_hbm.at[pl.ds(base, GS)])

        issue(0, 0)

        def body(i, carry):
            g = 2 * i
            issue(g + 1, 1)
            drain(g, 0)
            issue(g + 2, 0)
            drain(g + 1, 1)
            return carry

        lax.fori_loop(0, n_groups // 2 - 1, body, 0)
        g_last = n_groups - 1
        issue(g_last, 1)
        drain(g_last - 1, 0)
        drain(g_last, 1)

    return k(x2, table)


def kernel(x, table):
    B, H = x.shape
    D = table.shape[1]
    N = B * H
    info = plsc.get_sparse_core_info()
    NC, NS = info.num_cores, info.num_subcores
    assert N % (NC * NS * GS) == 0
    x2 = x.astype(jnp.int32).reshape(N // ROW, ROW)
    out = _embed(x2, table, N, NC, NS)
    return out.reshape(B, H, D)


# speedup vs baseline: 1.6358x; 1.2394x over previous
import functools

import jax
import jax.numpy as jnp
from jax import lax
from jax.experimental import layout as jlayout
from jax.experimental import pallas as pl
from jax.experimental.pallas import tpu as pltpu
from jax.experimental.pallas import tpu_sc as plsc

ROW = 128
GROUP_ROWS = 4
GS = ROW * GROUP_ROWS
NBUF = 2


@functools.partial(jax.jit, static_argnums=(2, 3, 4))
def _embed(x2, table, N, NC, NS):
    D = table.shape[1]
    NW = NC * NS
    per_w = N // NW
    n_groups = per_w // GS
    assert n_groups % 2 == 0 and n_groups >= 4
    mesh = plsc.VectorSubcoreMesh(core_axis_name="c", subcore_axis_name="s")

    @functools.partial(
        pl.kernel,
        mesh=mesh,
        out_type=jax.ShapeDtypeStruct((N, D), jnp.float32),
        compiler_params=pltpu.CompilerParams(use_tc_tiling_on_sc=False),
        scratch_types=[
            pltpu.VMEM((NBUF, GROUP_ROWS, ROW), jnp.int32),
            pltpu.VMEM((NBUF, GS, D), jnp.float32),
            pltpu.SemaphoreType.DMA,
            pltpu.SemaphoreType.DMA,
        ],
    )
    def k(x_hbm, table_hbm, out_hbm, idx_v, rows_v, sem0, sem1):
        wid = lax.axis_index("s") * NC + lax.axis_index("c")
        wb = wid * per_w
        sems = [sem0, sem1]

        def issue(g, b):
            base = pl.multiple_of(wb + g * GS, GS)
            row0 = pl.multiple_of(base // ROW, GROUP_ROWS)
            pltpu.sync_copy(x_hbm.at[pl.ds(row0, GROUP_ROWS)], idx_v.at[b])
            for j in range(GROUP_ROWS):
                pltpu.async_copy(
                    table_hbm.at[idx_v.at[b, j]],
                    rows_v.at[b, pl.ds(j * ROW, ROW)],
                    sems[b],
                )

        def drain(g, b):
            pltpu.make_async_copy(
                out_hbm.at[pl.ds(0, GS)], rows_v.at[b], sems[b]
            ).wait()
            base = pl.multiple_of(wb + g * GS, GS)
            pltpu.sync_copy(rows_v.at[b], out_hbm.at[pl.ds(base, GS)])

        issue(0, 0)

        def body(i, carry):
            g = 2 * i
            issue(g + 1, 1)
            drain(g, 0)
            issue(g + 2, 0)
            drain(g + 1, 1)
            return carry

        lax.fori_loop(0, n_groups // 2 - 1, body, 0)
        g_last = n_groups - 1
        issue(g_last, 1)
        drain(g_last - 1, 0)
        drain(g_last, 1)

    return k(x2, table)


def kernel(x, table):
    B, H = x.shape
    D = table.shape[1]
    N = B * H
    info = plsc.get_sparse_core_info()
    NC, NS = info.num_cores, info.num_subcores
    assert N % (NC * NS * GS) == 0
    x2 = x.astype(jnp.int32).reshape(N // ROW, ROW)
    table = jlayout.with_layout_constraint(
        table, jlayout.Layout((0, 1), ((8,),)))
    out = _embed(x2, table, N, NC, NS)
    return out.reshape(B, H, D)
